# Initial kernel scaffold; baseline (speedup 1.0000x reference)
import functools

import jax
import jax.numpy as jnp
from jax import lax
from jax.experimental import pallas as pl
from jax.experimental.pallas import tpu as pltpu
from jax.experimental.pallas import tpu_sc as plsc

N = 10000
E = 320000
D = 128
H = 128
C = 64

NC = 2
NS = 16
NW = NC * NS
CHUNK = 80
E_PER_W = E // NW
N_CHUNKS = E_PER_W // CHUNK
ROWS_PER_TILE = N // NS


def _make_edge_agg(F):
    mesh = plsc.VectorSubcoreMesh(core_axis_name="c", subcore_axis_name="s")

    @functools.partial(
        pl.kernel,
        mesh=mesh,
        out_type=jax.ShapeDtypeStruct((NC, N, F), jnp.float32),
        scratch_types=[
            pltpu.VMEM((CHUNK,), jnp.int32),
            pltpu.VMEM((CHUNK,), jnp.int32),
            pltpu.VMEM((CHUNK,), jnp.float32),
            pltpu.VMEM((CHUNK, F), jnp.float32),
            pltpu.VMEM_SHARED((N, F), jnp.float32),
            pltpu.SemaphoreType.DMA,
        ],
    )
    def k(hs_hbm, row_hbm, col_hbm, ew_hbm, zeros_hbm, out_hbm,
          ridx_v, cidx_v, ew_v, rows_v, acc_sh, sem):
        cid = lax.axis_index("c")
        sid = lax.axis_index("s")
        wid = sid * NC + cid
        pltpu.sync_copy(
            zeros_hbm.at[pl.ds(sid * ROWS_PER_TILE, ROWS_PER_TILE)],
            acc_sh.at[pl.ds(sid * ROWS_PER_TILE, ROWS_PER_TILE)])
        plsc.subcore_barrier()

        base_w = wid * E_PER_W

        def body(i, carry):
            base = base_w + i * CHUNK
            pltpu.sync_copy(row_hbm.at[pl.ds(base, CHUNK)], ridx_v)
            pltpu.sync_copy(col_hbm.at[pl.ds(base, CHUNK)], cidx_v)
            pltpu.sync_copy(ew_hbm.at[pl.ds(base, CHUNK)], ew_v)
            pltpu.async_copy(hs_hbm.at[ridx_v], rows_v, sem).wait()

            def scale(e, c2):
                s = ew_v[e]
                for j in range(F // 16):
                    rows_v[e, pl.ds(j * 16, 16)] = rows_v[e, pl.ds(j * 16, 16)] * s
                return c2

            lax.fori_loop(0, CHUNK, scale, 0)
            pltpu.sync_copy(rows_v, acc_sh.at[cidx_v], add=True)
            return carry

        lax.fori_loop(0, N_CHUNKS, body, 0)
        plsc.subcore_barrier()
        pltpu.sync_copy(
            acc_sh.at[pl.ds(sid * ROWS_PER_TILE, ROWS_PER_TILE)],
            out_hbm.at[cid, pl.ds(sid * ROWS_PER_TILE, ROWS_PER_TILE)])

    return k


_edge_agg = {F: _make_edge_agg(F) for F in (16, H, C)}

RB = 2000
GRID = N // RB


def _dis_block(degp_ref):
    deg = degp_ref[0, :, 0:1] + degp_ref[1, :, 0:1] + 1.0
    return jnp.where(deg > 0, lax.rsqrt(deg), 0.0)


def _tc_first(x_ref, w1_ref, degp_ref, hs_ref):
    dis = _dis_block(degp_ref)
    h = jnp.dot(x_ref[...], w1_ref[...], preferred_element_type=jnp.float32)
    hs_ref[...] = h * dis


def _tc_mid(u_ref, hs1_ref, degp_ref, b1_ref, w2_ref, hs2_ref):
    dis = _dis_block(degp_ref)
    agg = (u_ref[0] + u_ref[1] + hs1_ref[...]) * dis + b1_ref[...]
    hrelu = jnp.maximum(agg, 0.0)
    h2 = jnp.dot(hrelu, w2_ref[...], preferred_element_type=jnp.float32)
    hs2_ref[...] = h2 * dis


def _tc_last(u_ref, hs2_ref, degp_ref, b2_ref, out_ref, z_ref):
    dis = _dis_block(degp_ref)
    z = (u_ref[0] + u_ref[1] + hs2_ref[...]) * dis + b2_ref[...]
    m = jnp.max(z, axis=1, keepdims=True)
    ez = jnp.exp(z - m)
    lse = jnp.log(jnp.sum(ez, axis=1, keepdims=True))
    z_ref[...] = z
    out_ref[...] = z - m - lse


def _row_spec(F):
    return pl.BlockSpec((RB, F), lambda i: (i, 0))


def _full_spec(shape):
    return pl.BlockSpec(shape, lambda i: tuple(0 for _ in shape))


_DEGP_SPEC = pl.BlockSpec((NC, RB, 16), lambda i: (0, i, 0))


def kernel(x, edge_index, edge_weight, W1, b1, W2, b2):
    row = edge_index[0]
    col = edge_index[1]
    ones16 = jnp.ones((N, 16), jnp.float32)
    z16 = jnp.zeros((N, 16), jnp.float32)
    zH = jnp.zeros((N, H), jnp.float32)
    zC = jnp.zeros((N, C), jnp.float32)

    degp = _edge_agg[16](ones16, row, col, edge_weight, z16)

    hs1 = pl.pallas_call(
        _tc_first,
        grid=(GRID,),
        in_specs=[_row_spec(D), _full_spec((D, H)), _DEGP_SPEC],
        out_specs=_row_spec(H),
        out_shape=jax.ShapeDtypeStruct((N, H), jnp.float32),
    )(x, W1, degp)

    u1 = _edge_agg[H](hs1, row, col, edge_weight, zH)

    hs2 = pl.pallas_call(
        _tc_mid,
        grid=(GRID,),
        in_specs=[pl.BlockSpec((NC, RB, H), lambda i: (0, i, 0)),
                  _row_spec(H), _DEGP_SPEC, _full_spec((1, H)),
                  _full_spec((H, C))],
        out_specs=_row_spec(C),
        out_shape=jax.ShapeDtypeStruct((N, C), jnp.float32),
    )(u1, hs1, degp, b1.reshape(1, H), W2)

    u2 = _edge_agg[C](hs2, row, col, edge_weight, zC)

    out, z = pl.pallas_call(
        _tc_last,
        grid=(GRID,),
        in_specs=[pl.BlockSpec((NC, RB, C), lambda i: (0, i, 0)),
                  _row_spec(C), _DEGP_SPEC, _full_spec((1, C))],
        out_specs=[_row_spec(C), _row_spec(C)],
        out_shape=[jax.ShapeDtypeStruct((N, C), jnp.float32),
                   jax.ShapeDtypeStruct((N, C), jnp.float32)],
    )(u2, hs2, degp, b2.reshape(1, C))

    return (out, z, z)


# speedup vs baseline: 9.0070x; 9.0070x over previous
import functools

import jax
import jax.numpy as jnp
from jax import lax
from jax.experimental import pallas as pl
from jax.experimental.pallas import tpu as pltpu
from jax.experimental.pallas import tpu_sc as plsc

N = 10000
E = 320000
D = 128
H = 128
C = 64

NC = 2
NS = 16
NW = NC * NS
CHUNK = 80
E_PER_W = E // NW
N_CHUNKS = E_PER_W // CHUNK
NPAD = 10240
ROWS_PER_TILE = NPAD // NS


def _make_edge_agg(F):
    mesh = plsc.VectorSubcoreMesh(core_axis_name="c", subcore_axis_name="s")

    @functools.partial(
        pl.kernel,
        mesh=mesh,
        compiler_params=pltpu.CompilerParams(use_tc_tiling_on_sc=False),
        out_type=jax.ShapeDtypeStruct((NC, NPAD, F), jnp.float32),
        scratch_types=[
            pltpu.VMEM((CHUNK,), jnp.int32),
            pltpu.VMEM((CHUNK,), jnp.int32),
            pltpu.VMEM((CHUNK,), jnp.float32),
            pltpu.VMEM((CHUNK, F), jnp.float32),
            pltpu.VMEM_SHARED((NPAD, F), jnp.float32),
            pltpu.SemaphoreType.DMA,
        ],
    )
    def k(hs_hbm, row_hbm, col_hbm, ew_hbm, zeros_hbm, out_hbm,
          ridx_v, cidx_v, ew_v, rows_v, acc_sh, sem):
        cid = lax.axis_index("c")
        sid = lax.axis_index("s")
        wid = sid * NC + cid
        pltpu.sync_copy(
            zeros_hbm.at[pl.ds(sid * ROWS_PER_TILE, ROWS_PER_TILE)],
            acc_sh.at[pl.ds(sid * ROWS_PER_TILE, ROWS_PER_TILE)])
        plsc.subcore_barrier()

        base_w = wid * E_PER_W

        def body(i, carry):
            base = base_w + i * CHUNK
            pltpu.sync_copy(row_hbm.at[pl.ds(base, CHUNK)], ridx_v)
            pltpu.sync_copy(col_hbm.at[pl.ds(base, CHUNK)], cidx_v)
            pltpu.sync_copy(ew_hbm.at[pl.ds(base, CHUNK)], ew_v)
            pltpu.async_copy(hs_hbm.at[ridx_v], rows_v, sem).wait()

            def scale(g, c2):
                ws = ew_v[pl.ds(g * 16, 16)]
                for l in range(16):
                    s = ws[l]
                    e = g * 16 + l
                    for j in range(F // 16):
                        rows_v[e, pl.ds(j * 16, 16)] = (
                            rows_v[e, pl.ds(j * 16, 16)] * s)
                return c2

            lax.fori_loop(0, CHUNK // 16, scale, 0)
            pltpu.sync_copy(rows_v, acc_sh.at[cidx_v], add=True)
            return carry

        lax.fori_loop(0, N_CHUNKS, body, 0)
        plsc.subcore_barrier()
        pltpu.sync_copy(
            acc_sh.at[pl.ds(sid * ROWS_PER_TILE, ROWS_PER_TILE)],
            out_hbm.at[cid, pl.ds(sid * ROWS_PER_TILE, ROWS_PER_TILE)])

    return k


_edge_agg = {F: _make_edge_agg(F) for F in (16, H, C)}

RB = 2000
GRID = N // RB


def _dis_block(degp_ref):
    deg = degp_ref[0, :, 0:1] + degp_ref[1, :, 0:1] + 1.0
    return jnp.where(deg > 0, lax.rsqrt(deg), 0.0)


def _tc_first(x_ref, w1_ref, degp_ref, hs_ref):
    dis = _dis_block(degp_ref)
    h = jnp.dot(x_ref[...], w1_ref[...], preferred_element_type=jnp.float32)
    hs_ref[...] = h * dis


def _tc_mid(u_ref, hs1_ref, degp_ref, b1_ref, w2_ref, hs2_ref):
    dis = _dis_block(degp_ref)
    agg = (u_ref[0] + u_ref[1] + hs1_ref[...]) * dis + b1_ref[...]
    hrelu = jnp.maximum(agg, 0.0)
    h2 = jnp.dot(hrelu, w2_ref[...], preferred_element_type=jnp.float32)
    hs2_ref[...] = h2 * dis


def _tc_last(u_ref, hs2_ref, degp_ref, b2_ref, out_ref, z_ref):
    dis = _dis_block(degp_ref)
    z = (u_ref[0] + u_ref[1] + hs2_ref[...]) * dis + b2_ref[...]
    m = jnp.max(z, axis=1, keepdims=True)
    ez = jnp.exp(z - m)
    lse = jnp.log(jnp.sum(ez, axis=1, keepdims=True))
    z_ref[...] = z
    out_ref[...] = z - m - lse


def _row_spec(F):
    return pl.BlockSpec((RB, F), lambda i: (i, 0))


def _full_spec(shape):
    return pl.BlockSpec(shape, lambda i: tuple(0 for _ in shape))


_DEGP_SPEC = pl.BlockSpec((NC, RB, 16), lambda i: (0, i, 0))


def kernel(x, edge_index, edge_weight, W1, b1, W2, b2):
    row = edge_index[0]
    col = edge_index[1]
    ones16 = jnp.ones((N, 16), jnp.float32)
    z16 = jnp.zeros((NPAD, 16), jnp.float32)
    zH = jnp.zeros((NPAD, H), jnp.float32)
    zC = jnp.zeros((NPAD, C), jnp.float32)

    degp = _edge_agg[16](ones16, row, col, edge_weight, z16)

    hs1 = pl.pallas_call(
        _tc_first,
        grid=(GRID,),
        in_specs=[_row_spec(D), _full_spec((D, H)), _DEGP_SPEC],
        out_specs=_row_spec(H),
        out_shape=jax.ShapeDtypeStruct((N, H), jnp.float32),
    )(x, W1, degp)

    u1 = _edge_agg[H](hs1, row, col, edge_weight, zH)

    hs2 = pl.pallas_call(
        _tc_mid,
        grid=(GRID,),
        in_specs=[pl.BlockSpec((NC, RB, H), lambda i: (0, i, 0)),
                  _row_spec(H), _DEGP_SPEC, _full_spec((1, H)),
                  _full_spec((H, C))],
        out_specs=_row_spec(C),
        out_shape=jax.ShapeDtypeStruct((N, C), jnp.float32),
    )(u1, hs1, degp, b1.reshape(1, H), W2)

    u2 = _edge_agg[C](hs2, row, col, edge_weight, zC)

    out, z = pl.pallas_call(
        _tc_last,
        grid=(GRID,),
        in_specs=[pl.BlockSpec((NC, RB, C), lambda i: (0, i, 0)),
                  _row_spec(C), _DEGP_SPEC, _full_spec((1, C))],
        out_specs=[_row_spec(C), _row_spec(C)],
        out_shape=[jax.ShapeDtypeStruct((N, C), jnp.float32),
                   jax.ShapeDtypeStruct((N, C), jnp.float32)],
    )(u2, hs2, degp, b2.reshape(1, C))

    return (out, z, z)


# speedup vs baseline: 13.2366x; 1.4696x over previous
import functools

import jax
import jax.numpy as jnp
from jax import lax
from jax.experimental import pallas as pl
from jax.experimental.pallas import tpu as pltpu
from jax.experimental.pallas import tpu_sc as plsc

N = 10000
E = 320000
D = 128
H = 128
C = 64

NC = 2
NS = 16
NW = NC * NS
CHUNK = 80
E_PER_W = E // NW
N_CHUNKS = E_PER_W // CHUNK
NBUF = 5
OUTER = N_CHUNKS // NBUF
NPAD = 10240
RPT = NPAD // NS
_SC_PARAMS = pltpu.CompilerParams(use_tc_tiling_on_sc=False)


def _zero_acc_slice(buf, acc_sh, sid, F):
    zeros16 = jnp.zeros((16,), jnp.float32)

    def zbody(e, c):
        for j in range(F // 16):
            buf[e, pl.ds(j * 16, 16)] = zeros16
        return c

    lax.fori_loop(0, CHUNK, zbody, 0)
    for r in range(RPT // CHUNK):
        pltpu.sync_copy(buf, acc_sh.at[pl.ds(sid * RPT + r * CHUNK, CHUNK)])


def _scale_rows(buf, ew_all, i, F):
    def scale(g, c):
        ws = ew_all[i, pl.ds(g * 16, 16)]
        for l in range(16):
            s = ws[l]
            e = g * 16 + l
            for j in range(F // 16):
                buf[e, pl.ds(j * 16, 16)] = buf[e, pl.ds(j * 16, 16)] * s
        return c

    lax.fori_loop(0, CHUNK // 16, scale, 0)


def _make_edge_agg(F):
    mesh = plsc.VectorSubcoreMesh(core_axis_name="c", subcore_axis_name="s")

    @functools.partial(
        pl.kernel,
        mesh=mesh,
        compiler_params=_SC_PARAMS,
        out_type=jax.ShapeDtypeStruct((NC, NPAD, F), jnp.float32),
        scratch_types=[
            pltpu.VMEM((N_CHUNKS, CHUNK), jnp.int32),
            pltpu.VMEM((N_CHUNKS, CHUNK), jnp.int32),
            pltpu.VMEM((N_CHUNKS, CHUNK), jnp.float32),
            pltpu.VMEM((CHUNK, F), jnp.float32),
            pltpu.VMEM((CHUNK, F), jnp.float32),
            pltpu.VMEM((CHUNK, F), jnp.float32),
            pltpu.VMEM((CHUNK, F), jnp.float32),
            pltpu.VMEM((CHUNK, F), jnp.float32),
            pltpu.SemaphoreType.DMA,
            pltpu.SemaphoreType.DMA,
            pltpu.SemaphoreType.DMA,
            pltpu.SemaphoreType.DMA,
            pltpu.SemaphoreType.DMA,
            pltpu.SemaphoreType.DMA,
            pltpu.SemaphoreType.DMA,
            pltpu.SemaphoreType.DMA,
            pltpu.SemaphoreType.DMA,
            pltpu.SemaphoreType.DMA,
            pltpu.VMEM_SHARED((NPAD, F), jnp.float32),
        ],
    )
    def k(hs_hbm, row2_hbm, col2_hbm, ew2_hbm, out_hbm,
          ridx_all, cidx_all, ew_all, b0, b1, b2, b3, b4,
          g0, g1, g2, g3, g4, s0, s1, s2, s3, s4, acc_sh):
        bufs = (b0, b1, b2, b3, b4)
        gsems = (g0, g1, g2, g3, g4)
        ssems = (s0, s1, s2, s3, s4)
        cid = lax.axis_index("c")
        sid = lax.axis_index("s")
        wid = sid * NC + cid

        pltpu.sync_copy(row2_hbm.at[pl.ds(wid * N_CHUNKS, N_CHUNKS)], ridx_all)
        pltpu.sync_copy(col2_hbm.at[pl.ds(wid * N_CHUNKS, N_CHUNKS)], cidx_all)
        pltpu.sync_copy(ew2_hbm.at[pl.ds(wid * N_CHUNKS, N_CHUNKS)], ew_all)
        _zero_acc_slice(b0, acc_sh, sid, F)
        plsc.subcore_barrier()

        for b in range(NBUF):
            pltpu.async_copy(hs_hbm.at[ridx_all.at[b]], bufs[b], gsems[b])

        def body(kk, carry):
            base = kk * NBUF
            descs = []
            for b in range(NBUF):
                i = base + b
                pltpu.make_async_copy(
                    hs_hbm.at[ridx_all.at[i]], bufs[b], gsems[b]).wait()
                _scale_rows(bufs[b], ew_all, i, F)
                descs.append(pltpu.async_copy(
                    bufs[b], acc_sh.at[cidx_all.at[i]], ssems[b], add=True))
            for b in range(NBUF):
                descs[b].wait()
                def _next_gather(b=b):
                    pltpu.async_copy(
                        hs_hbm.at[ridx_all.at[base + NBUF + b]], bufs[b],
                        gsems[b])

                pl.when(kk < OUTER - 1)(_next_gather)
            return carry

        lax.fori_loop(0, OUTER, body, 0)
        plsc.subcore_barrier()
        pltpu.sync_copy(acc_sh.at[pl.ds(sid * RPT, RPT)],
                        out_hbm.at[cid, pl.ds(sid * RPT, RPT)])

    return k


def _make_deg():
    F = 16
    mesh = plsc.VectorSubcoreMesh(core_axis_name="c", subcore_axis_name="s")

    @functools.partial(
        pl.kernel,
        mesh=mesh,
        compiler_params=_SC_PARAMS,
        out_type=jax.ShapeDtypeStruct((NC, NPAD, F), jnp.float32),
        scratch_types=[
            pltpu.VMEM((N_CHUNKS, CHUNK), jnp.int32),
            pltpu.VMEM((N_CHUNKS, CHUNK), jnp.float32),
            pltpu.VMEM((CHUNK, F), jnp.float32),
            pltpu.VMEM((CHUNK, F), jnp.float32),
            pltpu.VMEM((CHUNK, F), jnp.float32),
            pltpu.VMEM((CHUNK, F), jnp.float32),
            pltpu.VMEM((CHUNK, F), jnp.float32),
            pltpu.SemaphoreType.DMA,
            pltpu.SemaphoreType.DMA,
            pltpu.SemaphoreType.DMA,
            pltpu.SemaphoreType.DMA,
            pltpu.SemaphoreType.DMA,
            pltpu.VMEM_SHARED((NPAD, F), jnp.float32),
        ],
    )
    def k(col2_hbm, ew2_hbm, out_hbm, cidx_all, ew_all,
          b0, b1, b2, b3, b4, s0, s1, s2, s3, s4, acc_sh):
        bufs = (b0, b1, b2, b3, b4)
        ssems = (s0, s1, s2, s3, s4)
        cid = lax.axis_index("c")
        sid = lax.axis_index("s")
        wid = sid * NC + cid

        pltpu.sync_copy(col2_hbm.at[pl.ds(wid * N_CHUNKS, N_CHUNKS)], cidx_all)
        pltpu.sync_copy(ew2_hbm.at[pl.ds(wid * N_CHUNKS, N_CHUNKS)], ew_all)
        _zero_acc_slice(b0, acc_sh, sid, F)
        plsc.subcore_barrier()

        def body(kk, carry):
            base = kk * NBUF
            descs = []
            for b in range(NBUF):
                i = base + b

                def build(g, c, _b=b, _i=i):
                    ws = ew_all[_i, pl.ds(g * 16, 16)]
                    for l in range(16):
                        bufs[_b][g * 16 + l, pl.ds(0, 16)] = jnp.full(
                            (16,), ws[l], jnp.float32)
                    return c

                lax.fori_loop(0, CHUNK // 16, build, 0)
                descs.append(pltpu.async_copy(
                    bufs[b], acc_sh.at[cidx_all.at[i]], ssems[b], add=True))
            for b in range(NBUF):
                descs[b].wait()
            return carry

        lax.fori_loop(0, OUTER, body, 0)
        plsc.subcore_barrier()
        pltpu.sync_copy(acc_sh.at[pl.ds(sid * RPT, RPT)],
                        out_hbm.at[cid, pl.ds(sid * RPT, RPT)])

    return k


_edge_agg64 = _make_edge_agg(64)
_deg_agg = _make_deg()

RB = 2000
GRID = N // RB


def _dis_block(degp_ref):
    deg = degp_ref[0, :, 0:1] + degp_ref[1, :, 0:1] + 1.0
    return jnp.where(deg > 0, lax.rsqrt(deg), 0.0)


def _tc_first(x_ref, w1_ref, degp_ref, hsl_ref, hsr_ref):
    dis = _dis_block(degp_ref)
    h = jnp.dot(x_ref[...], w1_ref[...], preferred_element_type=jnp.float32)
    hs = h * dis
    hsl_ref[...] = hs[:, :64]
    hsr_ref[...] = hs[:, 64:]


def _tc_mid(ul_ref, ur_ref, hsl_ref, hsr_ref, degp_ref, b1_ref, w2_ref,
            hs2_ref):
    dis = _dis_block(degp_ref)
    aggl = (ul_ref[0] + ul_ref[1] + hsl_ref[...]) * dis + b1_ref[:, :64]
    aggr = (ur_ref[0] + ur_ref[1] + hsr_ref[...]) * dis + b1_ref[:, 64:]
    hl = jnp.maximum(aggl, 0.0)
    hr = jnp.maximum(aggr, 0.0)
    h2 = (jnp.dot(hl, w2_ref[0:64], preferred_element_type=jnp.float32)
          + jnp.dot(hr, w2_ref[64:128], preferred_element_type=jnp.float32))
    hs2_ref[...] = h2 * dis


def _tc_last(u_ref, hs2_ref, degp_ref, b2_ref, out_ref, z_ref):
    dis = _dis_block(degp_ref)
    z = (u_ref[0] + u_ref[1] + hs2_ref[...]) * dis + b2_ref[...]
    m = jnp.max(z, axis=1, keepdims=True)
    ez = jnp.exp(z - m)
    lse = jnp.log(jnp.sum(ez, axis=1, keepdims=True))
    z_ref[...] = z
    out_ref[...] = z - m - lse


def _row_spec(F):
    return pl.BlockSpec((RB, F), lambda i: (i, 0))


def _full_spec(shape):
    return pl.BlockSpec(shape, lambda i: tuple(0 for _ in shape))


_DEGP_SPEC = pl.BlockSpec((NC, RB, 16), lambda i: (0, i, 0))


def kernel(x, edge_index, edge_weight, W1, b1, W2, b2):
    row2 = edge_index[0].reshape(NW * N_CHUNKS, CHUNK)
    col2 = edge_index[1].reshape(NW * N_CHUNKS, CHUNK)
    ew2 = edge_weight.reshape(NW * N_CHUNKS, CHUNK)

    degp = _deg_agg(col2, ew2)

    hsl, hsr = pl.pallas_call(
        _tc_first,
        grid=(GRID,),
        in_specs=[_row_spec(D), _full_spec((D, H)), _DEGP_SPEC],
        out_specs=[_row_spec(64), _row_spec(64)],
        out_shape=[jax.ShapeDtypeStruct((N, 64), jnp.float32),
                   jax.ShapeDtypeStruct((N, 64), jnp.float32)],
    )(x, W1, degp)

    u1l = _edge_agg64(hsl, row2, col2, ew2)
    u1r = _edge_agg64(hsr, row2, col2, ew2)

    _U64_SPEC = pl.BlockSpec((NC, RB, 64), lambda i: (0, i, 0))
    hs2 = pl.pallas_call(
        _tc_mid,
        grid=(GRID,),
        in_specs=[_U64_SPEC, _U64_SPEC, _row_spec(64), _row_spec(64),
                  _DEGP_SPEC, _full_spec((1, H)), _full_spec((H, C))],
        out_specs=_row_spec(C),
        out_shape=jax.ShapeDtypeStruct((N, C), jnp.float32),
    )(u1l, u1r, hsl, hsr, degp, b1.reshape(1, H), W2)

    u2 = _edge_agg64(hs2, row2, col2, ew2)

    out, z = pl.pallas_call(
        _tc_last,
        grid=(GRID,),
        in_specs=[pl.BlockSpec((NC, RB, C), lambda i: (0, i, 0)),
                  _row_spec(C), _DEGP_SPEC, _full_spec((1, C))],
        out_specs=[_row_spec(C), _row_spec(C)],
        out_shape=[jax.ShapeDtypeStruct((N, C), jnp.float32),
                   jax.ShapeDtypeStruct((N, C), jnp.float32)],
    )(u2, hs2, degp, b2.reshape(1, C))

    return (out, z, z)


# speedup vs baseline: 30.6627x; 2.3165x over previous
import functools

import jax
import jax.numpy as jnp
from jax import lax
from jax.experimental import pallas as pl
from jax.experimental.pallas import tpu as pltpu
from jax.experimental.pallas import tpu_sc as plsc

N = 10000
E = 320000
D = 128
H = 128
C = 64

NC = 2
NS = 16
NW = NC * NS
CHUNK = 80
E_PER_W = E // NW
N_CHUNKS = E_PER_W // CHUNK
NBUF = 5
OUTER = N_CHUNKS // NBUF
NPAD = 10240
RPT = NPAD // NS
_SC_PARAMS = pltpu.CompilerParams(use_tc_tiling_on_sc=False)


def _zero_acc_slice(buf, acc_sh, sid, F):
    zeros16 = jnp.zeros((16,), jnp.float32)

    def zbody(e, c):
        for j in range(F // 16):
            buf[e, pl.ds(j * 16, 16)] = zeros16
        return c

    lax.fori_loop(0, CHUNK, zbody, 0)
    for r in range(RPT // CHUNK):
        pltpu.sync_copy(buf, acc_sh.at[pl.ds(sid * RPT + r * CHUNK, CHUNK)])


def _scale_rows(src, dst, ew_all, i, F):
    def scale(g, c):
        ws = ew_all[i, pl.ds(g * 16, 16)]
        for l in range(16):
            s = ws[l]
            e = g * 16 + l
            for j in range(F // 16):
                dst[e, pl.ds(j * 16, 16)] = src[e, pl.ds(j * 16, 16)] * s
        return c

    lax.fori_loop(0, CHUNK // 16, scale, 0)


def _make_edge_agg(F):
    mesh = plsc.VectorSubcoreMesh(core_axis_name="c", subcore_axis_name="s")

    @functools.partial(
        pl.kernel,
        mesh=mesh,
        compiler_params=_SC_PARAMS,
        out_type=jax.ShapeDtypeStruct((NC, NPAD, F), jnp.float32),
        scratch_types=[
            pltpu.VMEM((N_CHUNKS, CHUNK), jnp.int32),
            pltpu.VMEM((N_CHUNKS, CHUNK), jnp.int32),
            pltpu.VMEM((N_CHUNKS, CHUNK), jnp.float32),
            pltpu.VMEM((CHUNK, F), jnp.float32),
            pltpu.VMEM((CHUNK, F), jnp.float32),
            pltpu.VMEM((CHUNK, F), jnp.float32),
            pltpu.VMEM((CHUNK, F), jnp.float32),
            pltpu.VMEM((CHUNK, F), jnp.float32),
            pltpu.VMEM((CHUNK, F), jnp.float32),
            pltpu.VMEM((CHUNK, F), jnp.float32),
            pltpu.VMEM((CHUNK, F), jnp.float32),
            pltpu.VMEM((CHUNK, F), jnp.float32),
            pltpu.VMEM((CHUNK, F), jnp.float32),
            pltpu.SemaphoreType.DMA,
            pltpu.SemaphoreType.DMA,
            pltpu.SemaphoreType.DMA,
            pltpu.SemaphoreType.DMA,
            pltpu.SemaphoreType.DMA,
            pltpu.SemaphoreType.DMA,
            pltpu.SemaphoreType.DMA,
            pltpu.SemaphoreType.DMA,
            pltpu.SemaphoreType.DMA,
            pltpu.SemaphoreType.DMA,
            pltpu.VMEM_SHARED((NPAD, F), jnp.float32),
        ],
    )
    def k(hs_hbm, row2_hbm, col2_hbm, ew2_hbm, out_hbm,
          ridx_all, cidx_all, ew_all, a0, a1, a2, a3, a4,
          m0, m1, m2, m3, m4, g0, g1, g2, g3, g4, s0, s1, s2, s3, s4, acc_sh):
        gbufs = (a0, a1, a2, a3, a4)
        sbufs = (m0, m1, m2, m3, m4)
        gsems = (g0, g1, g2, g3, g4)
        ssems = (s0, s1, s2, s3, s4)
        cid = lax.axis_index("c")
        sid = lax.axis_index("s")
        wid = sid * NC + cid

        pltpu.sync_copy(row2_hbm.at[pl.ds(wid * N_CHUNKS, N_CHUNKS)], ridx_all)
        pltpu.sync_copy(col2_hbm.at[pl.ds(wid * N_CHUNKS, N_CHUNKS)], cidx_all)
        pltpu.sync_copy(ew2_hbm.at[pl.ds(wid * N_CHUNKS, N_CHUNKS)], ew_all)
        _zero_acc_slice(a0, acc_sh, sid, F)
        plsc.subcore_barrier()

        for b in range(NBUF):
            pltpu.async_copy(hs_hbm.at[ridx_all.at[b]], gbufs[b], gsems[b])

        def body(kk, carry):
            base = kk * NBUF
            for b in range(NBUF):
                i = base + b

                def _wait_prev_scatter(b=b, i=i):
                    pltpu.make_async_copy(
                        sbufs[b], acc_sh.at[cidx_all.at[i]], ssems[b]).wait()

                pl.when(kk > 0)(_wait_prev_scatter)
                pltpu.make_async_copy(
                    hs_hbm.at[ridx_all.at[i]], gbufs[b], gsems[b]).wait()
                _scale_rows(gbufs[b], sbufs[b], ew_all, i, F)
                pltpu.async_copy(
                    sbufs[b], acc_sh.at[cidx_all.at[i]], ssems[b], add=True)

                def _next_gather(b=b, base=base):
                    pltpu.async_copy(
                        hs_hbm.at[ridx_all.at[base + NBUF + b]], gbufs[b],
                        gsems[b])

                pl.when(kk < OUTER - 1)(_next_gather)
            return carry

        lax.fori_loop(0, OUTER, body, 0)
        for b in range(NBUF):
            pltpu.make_async_copy(
                sbufs[b],
                acc_sh.at[cidx_all.at[N_CHUNKS - NBUF + b]], ssems[b]).wait()
        plsc.subcore_barrier()
        pltpu.sync_copy(acc_sh.at[pl.ds(sid * RPT, RPT)],
                        out_hbm.at[cid, pl.ds(sid * RPT, RPT)])

    return k


def _make_deg():
    F = 16
    mesh = plsc.VectorSubcoreMesh(core_axis_name="c", subcore_axis_name="s")

    @functools.partial(
        pl.kernel,
        mesh=mesh,
        compiler_params=_SC_PARAMS,
        out_type=jax.ShapeDtypeStruct((NC, NPAD, F), jnp.float32),
        scratch_types=[
            pltpu.VMEM((N_CHUNKS, CHUNK), jnp.int32),
            pltpu.VMEM((N_CHUNKS, CHUNK), jnp.float32),
            pltpu.VMEM((CHUNK, F), jnp.float32),
            pltpu.VMEM((CHUNK, F), jnp.float32),
            pltpu.VMEM((CHUNK, F), jnp.float32),
            pltpu.VMEM((CHUNK, F), jnp.float32),
            pltpu.VMEM((CHUNK, F), jnp.float32),
            pltpu.SemaphoreType.DMA,
            pltpu.SemaphoreType.DMA,
            pltpu.SemaphoreType.DMA,
            pltpu.SemaphoreType.DMA,
            pltpu.SemaphoreType.DMA,
            pltpu.VMEM_SHARED((NPAD, F), jnp.float32),
        ],
    )
    def k(col2_hbm, ew2_hbm, out_hbm, cidx_all, ew_all,
          b0, b1, b2, b3, b4, s0, s1, s2, s3, s4, acc_sh):
        bufs = (b0, b1, b2, b3, b4)
        ssems = (s0, s1, s2, s3, s4)
        cid = lax.axis_index("c")
        sid = lax.axis_index("s")
        wid = sid * NC + cid

        pltpu.sync_copy(col2_hbm.at[pl.ds(wid * N_CHUNKS, N_CHUNKS)], cidx_all)
        pltpu.sync_copy(ew2_hbm.at[pl.ds(wid * N_CHUNKS, N_CHUNKS)], ew_all)
        _zero_acc_slice(b0, acc_sh, sid, F)
        plsc.subcore_barrier()

        def body(kk, carry):
            base = kk * NBUF
            descs = []
            for b in range(NBUF):
                i = base + b

                def build(g, c, _b=b, _i=i):
                    ws = ew_all[_i, pl.ds(g * 16, 16)]
                    for l in range(16):
                        bufs[_b][g * 16 + l, pl.ds(0, 16)] = jnp.full(
                            (16,), ws[l], jnp.float32)
                    return c

                lax.fori_loop(0, CHUNK // 16, build, 0)
                descs.append(pltpu.async_copy(
                    bufs[b], acc_sh.at[cidx_all.at[i]], ssems[b], add=True))
            for b in range(NBUF):
                descs[b].wait()
            return carry

        lax.fori_loop(0, OUTER, body, 0)
        plsc.subcore_barrier()
        pltpu.sync_copy(acc_sh.at[pl.ds(sid * RPT, RPT)],
                        out_hbm.at[cid, pl.ds(sid * RPT, RPT)])

    return k


_edge_agg64 = _make_edge_agg(64)
_deg_agg = _make_deg()

RB = 2000
GRID = N // RB


def _dis_block(degp_ref):
    deg = degp_ref[0, :, 0:1] + degp_ref[1, :, 0:1] + 1.0
    return jnp.where(deg > 0, lax.rsqrt(deg), 0.0)


def _tc_first(x_ref, w1_ref, degp_ref, hsl_ref, hsr_ref):
    dis = _dis_block(degp_ref)
    h = jnp.dot(x_ref[...], w1_ref[...], preferred_element_type=jnp.float32)
    hs = h * dis
    hsl_ref[...] = hs[:, :64]
    hsr_ref[...] = hs[:, 64:]


def _tc_mid(ul_ref, ur_ref, hsl_ref, hsr_ref, degp_ref, b1_ref, w2_ref,
            hs2_ref):
    dis = _dis_block(degp_ref)
    aggl = (ul_ref[0] + ul_ref[1] + hsl_ref[...]) * dis + b1_ref[:, :64]
    aggr = (ur_ref[0] + ur_ref[1] + hsr_ref[...]) * dis + b1_ref[:, 64:]
    hl = jnp.maximum(aggl, 0.0)
    hr = jnp.maximum(aggr, 0.0)
    h2 = (jnp.dot(hl, w2_ref[0:64], preferred_element_type=jnp.float32)
          + jnp.dot(hr, w2_ref[64:128], preferred_element_type=jnp.float32))
    hs2_ref[...] = h2 * dis


def _tc_last(u_ref, hs2_ref, degp_ref, b2_ref, out_ref, z_ref):
    dis = _dis_block(degp_ref)
    z = (u_ref[0] + u_ref[1] + hs2_ref[...]) * dis + b2_ref[...]
    m = jnp.max(z, axis=1, keepdims=True)
    ez = jnp.exp(z - m)
    lse = jnp.log(jnp.sum(ez, axis=1, keepdims=True))
    z_ref[...] = z
    out_ref[...] = z - m - lse


def _row_spec(F):
    return pl.BlockSpec((RB, F), lambda i: (i, 0))


def _full_spec(shape):
    return pl.BlockSpec(shape, lambda i: tuple(0 for _ in shape))


_DEGP_SPEC = pl.BlockSpec((NC, RB, 16), lambda i: (0, i, 0))


def kernel(x, edge_index, edge_weight, W1, b1, W2, b2):
    row2 = edge_index[0].reshape(NW * N_CHUNKS, CHUNK)
    col2 = edge_index[1].reshape(NW * N_CHUNKS, CHUNK)
    ew2 = edge_weight.reshape(NW * N_CHUNKS, CHUNK)

    degp = _deg_agg(col2, ew2)

    hsl, hsr = pl.pallas_call(
        _tc_first,
        grid=(GRID,),
        in_specs=[_row_spec(D), _full_spec((D, H)), _DEGP_SPEC],
        out_specs=[_row_spec(64), _row_spec(64)],
        out_shape=[jax.ShapeDtypeStruct((N, 64), jnp.float32),
                   jax.ShapeDtypeStruct((N, 64), jnp.float32)],
    )(x, W1, degp)

    u1l = _edge_agg64(hsl, row2, col2, ew2)
    u1r = _edge_agg64(hsr, row2, col2, ew2)

    _U64_SPEC = pl.BlockSpec((NC, RB, 64), lambda i: (0, i, 0))
    hs2 = pl.pallas_call(
        _tc_mid,
        grid=(GRID,),
        in_specs=[_U64_SPEC, _U64_SPEC, _row_spec(64), _row_spec(64),
                  _DEGP_SPEC, _full_spec((1, H)), _full_spec((H, C))],
        out_specs=_row_spec(C),
        out_shape=jax.ShapeDtypeStruct((N, C), jnp.float32),
    )(u1l, u1r, hsl, hsr, degp, b1.reshape(1, H), W2)

    u2 = _edge_agg64(hs2, row2, col2, ew2)

    out, z = pl.pallas_call(
        _tc_last,
        grid=(GRID,),
        in_specs=[pl.BlockSpec((NC, RB, C), lambda i: (0, i, 0)),
                  _row_spec(C), _DEGP_SPEC, _full_spec((1, C))],
        out_specs=[_row_spec(C), _row_spec(C)],
        out_shape=[jax.ShapeDtypeStruct((N, C), jnp.float32),
                   jax.ShapeDtypeStruct((N, C), jnp.float32)],
    )(u2, hs2, degp, b2.reshape(1, C))

    return (out, z, z)
